# baseline (device time: 69865 ns/iter reference)
import jax
import jax.numpy as jnp
from jax import lax
from jax.experimental import pallas as pl
from jax.experimental.pallas import tpu as pltpu

N_DEV = 32


def kernel(x, w_mat, scale_x, scale_w):
    m_per, k = x.shape
    _, n = w_mat.shape
    n_blk = n // N_DEV
    m_out = m_per * N_DEV

    def body(x_ref, w_ref, sx_ref, sw_ref, out_ref,
             x8_buf, send_buf, recv_buf, send_sems, recv_sems):
        p = lax.axis_index("i")
        c = pl.program_id(0)
        q = lax.rem(p + c, N_DEV)

        @pl.when(c == 0)
        def _():
            barrier_sem = pltpu.get_barrier_semaphore()
            for nbr in range(N_DEV):
                @pl.when(nbr != p)
                def _(nbr=nbr):
                    pl.semaphore_signal(
                        barrier_sem, inc=1,
                        device_id=(nbr,), device_id_type=pl.DeviceIdType.MESH,
                    )
            pl.semaphore_wait(barrier_sem, N_DEV - 1)
            x8_buf[...] = x_ref[...].astype(jnp.float8_e5m2)

        s = sx_ref[0] * sw_ref[0]
        w8 = w_ref[...].astype(jnp.float8_e5m2)
        acc = jnp.dot(x8_buf[...], w8, preferred_element_type=jnp.float32)
        y = acc * s
        chunk = y / (1.0 + jnp.exp(-jnp.clip(y, -60.0, 60.0)))

        @pl.when(c == 0)
        def _():
            recv_buf[p] = chunk

        @pl.when(c != 0)
        def _():
            send_buf[c] = chunk
            rdma = pltpu.make_async_remote_copy(
                src_ref=send_buf.at[c],
                dst_ref=recv_buf.at[p],
                send_sem=send_sems.at[c],
                recv_sem=recv_sems.at[p],
                device_id=(q,),
                device_id_type=pl.DeviceIdType.MESH,
            )
            rdma.start()

        @pl.when(c == N_DEV - 1)
        def _():
            for r in range(1, N_DEV):
                send = pltpu.make_async_remote_copy(
                    src_ref=send_buf.at[r],
                    dst_ref=recv_buf.at[r],
                    send_sem=send_sems.at[r],
                    recv_sem=recv_sems.at[r],
                    device_id=(r,),
                    device_id_type=pl.DeviceIdType.MESH,
                )
                send.wait_send()
            for r in range(N_DEV):
                @pl.when(r != p)
                def _(r=r):
                    recv = pltpu.make_async_remote_copy(
                        src_ref=send_buf.at[r],
                        dst_ref=recv_buf.at[r],
                        send_sem=send_sems.at[r],
                        recv_sem=recv_sems.at[r],
                        device_id=(r,),
                        device_id_type=pl.DeviceIdType.MESH,
                    )
                    recv.wait_recv()
            out_ref[...] = recv_buf[...].reshape(m_out, n_blk)

    def w_index(c):
        return (0, lax.rem(lax.axis_index("i") + c, N_DEV))

    return pl.pallas_call(
        body,
        grid=(N_DEV,),
        in_specs=[
            pl.BlockSpec((m_per, k), lambda c: (0, 0)),
            pl.BlockSpec((k, n_blk), w_index),
            pl.BlockSpec(memory_space=pltpu.SMEM),
            pl.BlockSpec(memory_space=pltpu.SMEM),
        ],
        out_specs=pl.BlockSpec((m_out, n_blk), lambda c: (0, 0)),
        out_shape=jax.ShapeDtypeStruct((m_out, n_blk), jnp.float32),
        scratch_shapes=[
            pltpu.VMEM((m_per, k), jnp.float8_e5m2),
            pltpu.VMEM((N_DEV, m_per, n_blk), jnp.float32),
            pltpu.VMEM((N_DEV, m_per, n_blk), jnp.float32),
            pltpu.SemaphoreType.DMA((N_DEV,)),
            pltpu.SemaphoreType.DMA((N_DEV,)),
        ],
        compiler_params=pltpu.CompilerParams(collective_id=0),
    )(x, w_mat, scale_x, scale_w)


# device time: 61981 ns/iter; 1.1272x vs baseline; 1.1272x over previous
import jax
import jax.numpy as jnp
from jax import lax
from jax.experimental import pallas as pl
from jax.experimental.pallas import tpu as pltpu

N_DEV = 32
N_STEP = 8
CPS = N_DEV // N_STEP


def kernel(x, w_mat, scale_x, scale_w):
    m_per, k = x.shape
    _, n = w_mat.shape
    n_blk = n // N_DEV
    n_sup = n // N_STEP
    m_out = m_per * N_DEV

    def body(x_ref, w_ref, sx_ref, sw_ref, out_ref,
             x8_buf, send_buf, recv_buf, send_sems, recv_sems):
        p = lax.axis_index("i")
        t = pl.program_id(0)
        g = lax.rem(p // CPS + t, N_STEP)
        jloc = lax.rem(p, CPS)

        @pl.when(t == 0)
        def _():
            barrier_sem = pltpu.get_barrier_semaphore()
            for nbr in range(N_DEV):
                @pl.when(nbr != p)
                def _(nbr=nbr):
                    pl.semaphore_signal(
                        barrier_sem, inc=1,
                        device_id=(nbr,), device_id_type=pl.DeviceIdType.MESH,
                    )
            pl.semaphore_wait(barrier_sem, N_DEV - 1)
            x8_buf[...] = x_ref[...].astype(jnp.float8_e5m2)

        s = sx_ref[0] * sw_ref[0]
        w8 = w_ref[...].astype(jnp.float8_e5m2)
        acc = jnp.dot(x8_buf[...], w8, preferred_element_type=jnp.float32)
        y = acc * s
        sup = y / (1.0 + jnp.exp(-jnp.clip(y, -60.0, 60.0)))

        for j in range(CPS):
            chunk = sup[:, j * n_blk:(j + 1) * n_blk].astype(jnp.bfloat16)
            q = g * CPS + j
            slot = t * CPS + j

            @pl.when((t == 0) & (jloc == j))
            def _(chunk=chunk):
                recv_buf[p] = chunk

            @pl.when((t != 0) | (jloc != j))
            def _(chunk=chunk, q=q, slot=slot):
                send_buf[slot] = chunk
                rdma = pltpu.make_async_remote_copy(
                    src_ref=send_buf.at[slot],
                    dst_ref=recv_buf.at[p],
                    send_sem=send_sems.at[slot],
                    recv_sem=recv_sems.at[p],
                    device_id=(q,),
                    device_id_type=pl.DeviceIdType.MESH,
                )
                rdma.start()

        @pl.when(t == N_STEP - 1)
        def _():
            for r in range(N_DEV):
                @pl.when(r != jloc)
                def _(r=r):
                    send = pltpu.make_async_remote_copy(
                        src_ref=send_buf.at[r],
                        dst_ref=recv_buf.at[r],
                        send_sem=send_sems.at[r],
                        recv_sem=recv_sems.at[r],
                        device_id=(r,),
                        device_id_type=pl.DeviceIdType.MESH,
                    )
                    send.wait_send()
            for r in range(N_DEV):
                @pl.when(r != p)
                def _(r=r):
                    recv = pltpu.make_async_remote_copy(
                        src_ref=send_buf.at[r],
                        dst_ref=recv_buf.at[r],
                        send_sem=send_sems.at[r],
                        recv_sem=recv_sems.at[r],
                        device_id=(r,),
                        device_id_type=pl.DeviceIdType.MESH,
                    )
                    recv.wait_recv()
            out_ref[...] = recv_buf[...].reshape(m_out, n_blk).astype(jnp.float32)

    def w_index(t):
        return (0, lax.rem(lax.axis_index("i") // CPS + t, N_STEP))

    return pl.pallas_call(
        body,
        grid=(N_STEP,),
        in_specs=[
            pl.BlockSpec((m_per, k), lambda t: (0, 0)),
            pl.BlockSpec((k, n_sup), w_index),
            pl.BlockSpec(memory_space=pltpu.SMEM),
            pl.BlockSpec(memory_space=pltpu.SMEM),
        ],
        out_specs=pl.BlockSpec((m_out, n_blk), lambda t: (0, 0)),
        out_shape=jax.ShapeDtypeStruct((m_out, n_blk), jnp.float32),
        scratch_shapes=[
            pltpu.VMEM((m_per, k), jnp.float8_e5m2),
            pltpu.VMEM((N_DEV, m_per, n_blk), jnp.bfloat16),
            pltpu.VMEM((N_DEV, m_per, n_blk), jnp.bfloat16),
            pltpu.SemaphoreType.DMA((N_DEV,)),
            pltpu.SemaphoreType.DMA((N_DEV,)),
        ],
        compiler_params=pltpu.CompilerParams(
            collective_id=0, vmem_limit_bytes=56 * 1024 * 1024
        ),
    )(x, w_mat, scale_x, scale_w)


# device time: 53053 ns/iter; 1.3169x vs baseline; 1.1683x over previous
import jax
import jax.numpy as jnp
from jax import lax
from jax.experimental import pallas as pl
from jax.experimental.pallas import tpu as pltpu

N_DEV = 32
N_STEP = 16


def kernel(x, w_mat, scale_x, scale_w):
    m_per, k = x.shape
    _, n = w_mat.shape
    n_blk = n // N_DEV
    m_grp = 2 * m_per
    m_out = m_per * N_DEV

    def body(x_ref, w_ref, sx_ref, sw_ref, out_ref,
             x8_pair, send_buf, recv_buf,
             x_send_sem, x_recv_sem, send_sems, recv_sems):
        p = lax.axis_index("i")
        u = pl.program_id(0)
        a = p // 2
        b = lax.rem(p, 2)
        j = 16 * b + lax.rem(a + u, N_STEP)
        myslot = pl.ds(b * m_per, m_per)

        @pl.when(u == 0)
        def _():
            barrier_sem = pltpu.get_barrier_semaphore()
            for nbr in range(N_DEV):
                @pl.when(nbr != p)
                def _(nbr=nbr):
                    pl.semaphore_signal(
                        barrier_sem, inc=1,
                        device_id=(nbr,), device_id_type=pl.DeviceIdType.MESH,
                    )
            pl.semaphore_wait(barrier_sem, N_DEV - 1)
            x8_pair[myslot, :] = x_ref[...].astype(jnp.float8_e5m2)
            xch = pltpu.make_async_remote_copy(
                src_ref=x8_pair.at[myslot, :],
                dst_ref=x8_pair.at[myslot, :],
                send_sem=x_send_sem,
                recv_sem=x_recv_sem,
                device_id=(p + 1 - 2 * b,),
                device_id_type=pl.DeviceIdType.MESH,
            )
            xch.start()
            xch.wait_send()
            other = pl.ds((1 - b) * m_per, m_per)
            recv = pltpu.make_async_remote_copy(
                src_ref=x8_pair.at[other, :],
                dst_ref=x8_pair.at[other, :],
                send_sem=x_send_sem,
                recv_sem=x_recv_sem,
                device_id=(p + 1 - 2 * b,),
                device_id_type=pl.DeviceIdType.MESH,
            )
            recv.wait_recv()

        s = sx_ref[0] * sw_ref[0]
        w8 = w_ref[...].astype(jnp.float8_e5m2)
        acc = jnp.dot(x8_pair[...], w8, preferred_element_type=jnp.float32)
        y = acc * s
        tile = (y / (1.0 + jnp.exp(-jnp.clip(y, -60.0, 60.0)))).astype(
            jnp.bfloat16)

        @pl.when(j == p)
        def _():
            recv_buf[a] = tile

        @pl.when(j != p)
        def _():
            send_buf[u] = tile
            rdma = pltpu.make_async_remote_copy(
                src_ref=send_buf.at[u],
                dst_ref=recv_buf.at[a],
                send_sem=send_sems.at[u],
                recv_sem=recv_sems.at[a],
                device_id=(j,),
                device_id_type=pl.DeviceIdType.MESH,
            )
            rdma.start()

        @pl.when(u == N_STEP - 1)
        def _():
            for uu in range(N_STEP):
                ju = 16 * b + lax.rem(a + uu, N_STEP)

                @pl.when(ju != p)
                def _(uu=uu):
                    send = pltpu.make_async_remote_copy(
                        src_ref=send_buf.at[uu],
                        dst_ref=recv_buf.at[0],
                        send_sem=send_sems.at[uu],
                        recv_sem=recv_sems.at[0],
                        device_id=(p,),
                        device_id_type=pl.DeviceIdType.MESH,
                    )
                    send.wait_send()
            for r in range(N_STEP):
                sender = 2 * r + p // 16

                @pl.when(sender != p)
                def _(r=r):
                    recv = pltpu.make_async_remote_copy(
                        src_ref=send_buf.at[0],
                        dst_ref=recv_buf.at[r],
                        send_sem=send_sems.at[0],
                        recv_sem=recv_sems.at[r],
                        device_id=(p,),
                        device_id_type=pl.DeviceIdType.MESH,
                    )
                    recv.wait_recv()
            out_ref[...] = recv_buf[...].reshape(m_out, n_blk).astype(jnp.float32)

    def w_index(u):
        p = lax.axis_index("i")
        return (0, 16 * lax.rem(p, 2) + lax.rem(p // 2 + u, N_STEP))

    return pl.pallas_call(
        body,
        grid=(N_STEP,),
        in_specs=[
            pl.BlockSpec((m_per, k), lambda u: (0, 0)),
            pl.BlockSpec((k, n_blk), w_index),
            pl.BlockSpec(memory_space=pltpu.SMEM),
            pl.BlockSpec(memory_space=pltpu.SMEM),
        ],
        out_specs=pl.BlockSpec((m_out, n_blk), lambda u: (0, 0)),
        out_shape=jax.ShapeDtypeStruct((m_out, n_blk), jnp.float32),
        scratch_shapes=[
            pltpu.VMEM((m_grp, k), jnp.float8_e5m2),
            pltpu.VMEM((N_STEP, m_grp, n_blk), jnp.bfloat16),
            pltpu.VMEM((N_STEP, m_grp, n_blk), jnp.bfloat16),
            pltpu.SemaphoreType.DMA,
            pltpu.SemaphoreType.DMA,
            pltpu.SemaphoreType.DMA((N_STEP,)),
            pltpu.SemaphoreType.DMA((N_STEP,)),
        ],
        compiler_params=pltpu.CompilerParams(
            collective_id=0, vmem_limit_bytes=56 * 1024 * 1024
        ),
    )(x, w_mat, scale_x, scale_w)


# device time: 52808 ns/iter; 1.3230x vs baseline; 1.0046x over previous
import jax
import jax.numpy as jnp
from jax import lax
from jax.experimental import pallas as pl
from jax.experimental.pallas import tpu as pltpu

N_DEV = 32
N_STEP = 16


def kernel(x, w_mat, scale_x, scale_w):
    m_per, k = x.shape
    _, n = w_mat.shape
    n_blk = n // N_DEV
    m_grp = 2 * m_per
    m_out = m_per * N_DEV

    def body(x_ref, w_ref, sx_ref, sw_ref, out_ref,
             x8_pair, send_buf, recv_buf,
             x_send_sem, x_recv_sem, send_sems, recv_sems):
        p = lax.axis_index("i")
        u = pl.program_id(0)
        a = p // 2
        b = lax.rem(p, 2)
        j = 2 * lax.rem(a + u, N_STEP) + b
        myslot = pl.ds(b * m_per, m_per)

        @pl.when(u == 0)
        def _():
            barrier_sem = pltpu.get_barrier_semaphore()
            for nbr in range(N_DEV):
                @pl.when(nbr != p)
                def _(nbr=nbr):
                    pl.semaphore_signal(
                        barrier_sem, inc=1,
                        device_id=(nbr,), device_id_type=pl.DeviceIdType.MESH,
                    )
            pl.semaphore_wait(barrier_sem, N_DEV - 1)
            x8_pair[myslot, :] = x_ref[...].astype(jnp.float8_e5m2)
            xch = pltpu.make_async_remote_copy(
                src_ref=x8_pair.at[myslot, :],
                dst_ref=x8_pair.at[myslot, :],
                send_sem=x_send_sem,
                recv_sem=x_recv_sem,
                device_id=(p + 1 - 2 * b,),
                device_id_type=pl.DeviceIdType.MESH,
            )
            xch.start()
            xch.wait_send()
            other = pl.ds((1 - b) * m_per, m_per)
            recv = pltpu.make_async_remote_copy(
                src_ref=x8_pair.at[other, :],
                dst_ref=x8_pair.at[other, :],
                send_sem=x_send_sem,
                recv_sem=x_recv_sem,
                device_id=(p + 1 - 2 * b,),
                device_id_type=pl.DeviceIdType.MESH,
            )
            recv.wait_recv()

        s = sx_ref[0] * sw_ref[0]
        w8 = w_ref[...].astype(jnp.float8_e5m2)
        acc = jnp.dot(x8_pair[...], w8, preferred_element_type=jnp.float32)
        y = acc * s
        tile = (y / (1.0 + jnp.exp(-jnp.clip(y, -60.0, 60.0)))).astype(
            jnp.bfloat16)

        @pl.when(j == p)
        def _():
            recv_buf[a] = tile

        @pl.when(j != p)
        def _():
            send_buf[u] = tile
            rdma = pltpu.make_async_remote_copy(
                src_ref=send_buf.at[u],
                dst_ref=recv_buf.at[a],
                send_sem=send_sems.at[u],
                recv_sem=recv_sems.at[a],
                device_id=(j,),
                device_id_type=pl.DeviceIdType.MESH,
            )
            rdma.start()

        @pl.when(u == N_STEP - 1)
        def _():
            for uu in range(N_STEP):
                ju = 2 * lax.rem(a + uu, N_STEP) + b

                @pl.when(ju != p)
                def _(uu=uu):
                    send = pltpu.make_async_remote_copy(
                        src_ref=send_buf.at[uu],
                        dst_ref=recv_buf.at[0],
                        send_sem=send_sems.at[uu],
                        recv_sem=recv_sems.at[0],
                        device_id=(p,),
                        device_id_type=pl.DeviceIdType.MESH,
                    )
                    send.wait_send()
            for r in range(N_STEP):
                sender = 2 * r + lax.rem(p, 2)

                @pl.when(sender != p)
                def _(r=r):
                    recv = pltpu.make_async_remote_copy(
                        src_ref=send_buf.at[0],
                        dst_ref=recv_buf.at[r],
                        send_sem=send_sems.at[0],
                        recv_sem=recv_sems.at[r],
                        device_id=(p,),
                        device_id_type=pl.DeviceIdType.MESH,
                    )
                    recv.wait_recv()
            out_ref[...] = recv_buf[...].reshape(m_out, n_blk).astype(jnp.float32)

    def w_index(u):
        p = lax.axis_index("i")
        return (0, 2 * lax.rem(p // 2 + u, N_STEP) + lax.rem(p, 2))

    return pl.pallas_call(
        body,
        grid=(N_STEP,),
        in_specs=[
            pl.BlockSpec((m_per, k), lambda u: (0, 0)),
            pl.BlockSpec((k, n_blk), w_index),
            pl.BlockSpec(memory_space=pltpu.SMEM),
            pl.BlockSpec(memory_space=pltpu.SMEM),
        ],
        out_specs=pl.BlockSpec((m_out, n_blk), lambda u: (0, 0)),
        out_shape=jax.ShapeDtypeStruct((m_out, n_blk), jnp.float32),
        scratch_shapes=[
            pltpu.VMEM((m_grp, k), jnp.float8_e5m2),
            pltpu.VMEM((N_STEP, m_grp, n_blk), jnp.bfloat16),
            pltpu.VMEM((N_STEP, m_grp, n_blk), jnp.bfloat16),
            pltpu.SemaphoreType.DMA,
            pltpu.SemaphoreType.DMA,
            pltpu.SemaphoreType.DMA((N_STEP,)),
            pltpu.SemaphoreType.DMA((N_STEP,)),
        ],
        compiler_params=pltpu.CompilerParams(
            collective_id=0, vmem_limit_bytes=56 * 1024 * 1024
        ),
    )(x, w_mat, scale_x, scale_w)
